# baseline (speedup 1.0000x reference)
import functools

import jax
import jax.numpy as jnp
from jax import lax
from jax.experimental import pallas as pl
from jax.experimental.pallas import tpu as pltpu
from jax.experimental.pallas import tpu_sc as plsc

_F = 26
_V = 100000
_D = 32
_B = 16384
_NC = 2
_NS = 16
_NW = _NC * _NS
_BPW = _B // _NW
_IPW = _BPW * _F
_CHUNK = 16
_IPS = _CHUNK * _F
_STEPS = _BPW // _CHUNK
_NBUF = 4
_LANES = 16


def _body(x_hbm, t_hbm, out_hbm, idx_v, b0, b1, b2, b3, out_v,
          s0, s1, s2, s3):
    wid = lax.axis_index("s") * _NC + lax.axis_index("c")
    ibase = wid * _IPW

    pltpu.sync_copy(x_hbm.at[pl.ds(ibase, _IPW)], idx_v)

    lane = lax.iota(jnp.int32, _LANES)

    def _off(i, c):
        s = pl.ds(i * _LANES, _LANES)
        pos = i * _LANES + lane
        idx_v[s] = idx_v[s] + lax.rem(pos, _F) * _V
        return c

    lax.fori_loop(0, _IPW // _LANES, _off, 0)

    bufs = (b0, b1, b2, b3)
    sems = (s0, s1, s2, s3)

    def _issue(g, b):
        pltpu.async_copy(
            t_hbm.at[idx_v.at[pl.ds(g * _IPS, _IPS)]], bufs[b], sems[b])

    def _wait(g, b):
        pltpu.make_async_copy(
            t_hbm.at[idx_v.at[pl.ds(g * _IPS, _IPS)]], bufs[b],
            sems[b]).wait()

    def _process(g, b):
        buf = bufs[b]

        def _row(r, c):
            row = r * _F
            orow = g * _CHUNK + r
            a0 = buf[row, pl.ds(0, _LANES)]
            a1 = buf[row, pl.ds(_LANES, _LANES)]
            for f in range(1, _F):
                a0 = a0 + buf[row + f, pl.ds(0, _LANES)]
                a1 = a1 + buf[row + f, pl.ds(_LANES, _LANES)]
            out_v[orow, pl.ds(0, _LANES)] = a0
            out_v[orow, pl.ds(_LANES, _LANES)] = a1
            return c

        lax.fori_loop(0, _CHUNK, _row, 0)

    for b in range(_NBUF):
        _issue(b, b)

    def _outer(k, c):
        g0 = k * _NBUF
        for b in range(_NBUF):
            g = g0 + b
            _wait(g, b)
            _process(g, b)

            @pl.when(g + _NBUF < _STEPS)
            def _():
                _issue(g + _NBUF, b)
        return c

    lax.fori_loop(0, _STEPS // _NBUF, _outer, 0)

    pltpu.sync_copy(out_v, out_hbm.at[pl.ds(wid * _BPW, _BPW)])


@functools.partial(
    pl.kernel,
    out_type=jax.ShapeDtypeStruct((_B, _D), jnp.float32),
    mesh=plsc.VectorSubcoreMesh(core_axis_name="c", subcore_axis_name="s"),
    compiler_params=pltpu.CompilerParams(use_tc_tiling_on_sc=False),
    scratch_types=[
        pltpu.VMEM((_IPW,), jnp.int32),
        pltpu.VMEM((_IPS, _D), jnp.float32),
        pltpu.VMEM((_IPS, _D), jnp.float32),
        pltpu.VMEM((_IPS, _D), jnp.float32),
        pltpu.VMEM((_IPS, _D), jnp.float32),
        pltpu.VMEM((_BPW, _D), jnp.float32),
        pltpu.SemaphoreType.DMA,
        pltpu.SemaphoreType.DMA,
        pltpu.SemaphoreType.DMA,
        pltpu.SemaphoreType.DMA,
    ],
)
def _emb(x_hbm, t_hbm, out_hbm, *scratch):
    _body(x_hbm, t_hbm, out_hbm, *scratch)


def kernel(g, x, tables):
    xf = x.astype(jnp.int32).reshape(_B * _F)
    tf = tables.reshape(_F * _V, _D)
    return _emb(xf, tf)
